# baseline (speedup 1.0000x reference)
import functools

import jax
import jax.numpy as jnp
from jax import lax
from jax.experimental import pallas as pl
from jax.experimental.pallas import tpu as pltpu
from jax.experimental.pallas import tpu_sc as plsc

B = 8
H = W = 224
HW = H * W
S = 1024
CIN = 3
CMID = 128
COUT = 768
SRC = 64
R = 8
BLK = R * W
NB = H // R
NTPB = 4
CH = HW // NTPB
NCH = CH // 16


def _sc_body(seg_hbm, newseg_hbm, counts_hbm, ranks_hbm, histall_hbm,
             ids_v, hist16_v, hist4_v, cnt_v, ranks_v, outbuf_v):
    c = lax.axis_index("c")
    s = lax.axis_index("s")
    b = c * 4 + s // 4
    q = s % 4
    wid = c * 16 + s
    base_row = wid - q

    pltpu.sync_copy(seg_hbm.at[pl.ds(b * HW + q * CH, CH)], ids_v)

    def _zero(i, _):
        hist16_v[pl.ds(i * 16, 16)] = jnp.zeros((16,), jnp.int32)
        return 0
    lax.fori_loop(0, 1024, _zero, 0)

    lane_off = lax.iota(jnp.int32, 16) * S
    ones16 = jnp.ones((16,), jnp.int32)

    def _hist(j, _):
        ids = ids_v[pl.ds(j * 16, 16)]
        plsc.addupdate_scatter(hist16_v, [ids + lane_off], ones16)
        return 0
    lax.fori_loop(0, NCH, _hist, 0)

    def _fold(i, _):
        acc = hist16_v[pl.ds(i * 16, 16)]
        for r in range(1, 16):
            acc = acc + hist16_v[pl.ds(r * S + i * 16, 16)]
        cnt_v[pl.ds(i * 16, 16)] = acc
        return 0
    lax.fori_loop(0, S // 16, _fold, 0)

    pltpu.sync_copy(cnt_v, histall_hbm.at[pl.ds(wid * S, S)])
    plsc.subcore_barrier()

    @pl.when(q == 0)
    def _phase2():
        pltpu.sync_copy(histall_hbm.at[pl.ds(base_row * S, 4 * S)], hist4_v)

        def _ranks(i, run):
            sl = pl.ds(i * 16, 16)
            h = (hist4_v[pl.ds(i * 16, 16)]
                 + hist4_v[pl.ds(S + i * 16, 16)]
                 + hist4_v[pl.ds(2 * S + i * 16, 16)]
                 + hist4_v[pl.ds(3 * S + i * 16, 16)])
            pres = jnp.where(h > 0, 1, 0).astype(jnp.int32)
            cs = plsc.cumsum(pres)
            ranks_v[sl] = cs - pres + run
            cnt_v[sl] = h
            return run + jnp.sum(pres)
        lax.fori_loop(0, S // 16, _ranks, jnp.int32(0))

        pltpu.sync_copy(ranks_v, ranks_hbm.at[pl.ds(b * S, S)])
        pltpu.sync_copy(cnt_v, counts_hbm.at[pl.ds(b * S, S)])

    plsc.subcore_barrier()

    pltpu.sync_copy(ranks_hbm.at[pl.ds(b * S, S)], ranks_v)

    def _gather(j, _):
        ids = ids_v[pl.ds(j * 16, 16)]
        outbuf_v[pl.ds(j * 16, 16)] = plsc.load_gather(ranks_v, [ids])
        return 0
    lax.fori_loop(0, NCH, _gather, 0)

    pltpu.sync_copy(outbuf_v, newseg_hbm.at[pl.ds(b * HW + q * CH, CH)])


def _sc_reindex(seg_flat):
    f = pl.kernel(
        _sc_body,
        mesh=plsc.VectorSubcoreMesh(core_axis_name="c", subcore_axis_name="s"),
        compiler_params=pltpu.CompilerParams(needs_layout_passes=False),
        out_type=[
            jax.ShapeDtypeStruct((B * HW,), jnp.int32),
            jax.ShapeDtypeStruct((B * S,), jnp.int32),
            jax.ShapeDtypeStruct((B * S,), jnp.int32),
            jax.ShapeDtypeStruct((32 * S,), jnp.int32),
        ],
        scratch_types=[
            pltpu.VMEM((CH,), jnp.int32),
            pltpu.VMEM((16 * S,), jnp.int32),
            pltpu.VMEM((4 * S,), jnp.int32),
            pltpu.VMEM((S,), jnp.int32),
            pltpu.VMEM((S,), jnp.int32),
            pltpu.VMEM((CH,), jnp.int32),
        ],
    )
    return f(seg_flat)


def _pos_body(pos_ref, wh_ref, ww_ref, out_ref, posw_scr):
    nb = pl.program_id(0)

    @pl.when(nb == 0)
    def _():
        posw_scr[...] = lax.dot_general(
            pos_ref[...], ww_ref[...], (((2,), (0,)), ((), ())),
            preferred_element_type=jnp.float32)

    whb = jnp.broadcast_to(wh_ref[...][None, :, :], (CMID, R, SRC))
    out_ref[...] = lax.dot_general(
        whb, posw_scr[...], (((2,), (1,)), ((0,), (0,))),
        preferred_element_type=jnp.float32)


def _main_body(x_ref, seg_ref, pos_ref, wc_ref, cb_ref, acc_ref):
    nb = pl.program_id(0)

    @pl.when(nb == 0)
    def _():
        acc_ref[...] = jnp.zeros_like(acc_ref)

    posr = pos_ref[...]
    iota_s = lax.broadcasted_iota(jnp.int32, (S, BLK), 0)
    for b in range(B):
        xb = x_ref[b]
        feat = lax.dot_general(wc_ref[...], xb, (((0,), (0,)), ((), ())),
                               preferred_element_type=jnp.float32)
        v = jnp.maximum(feat + cb_ref[...], 0.0) + posr
        ids = jnp.broadcast_to(seg_ref[b], (S, BLK))
        oht = (ids == iota_s).astype(jnp.bfloat16)
        acc_ref[b] = acc_ref[b] + lax.dot_general(
            v.astype(jnp.bfloat16), oht, (((1,), (1,)), ((), ())),
            preferred_element_type=jnp.float32)


def _proj_body(acc_ref, cnt_ref, rank_ref, pw_ref, pb_ref, out_ref):
    cnt = cnt_ref[0, 0, :]
    inv = 1.0 / jnp.maximum(cnt.astype(jnp.float32), 1.0)
    mean = acc_ref[0] * inv[None, :]

    ranks = rank_ref[0, 0, :]
    present = cnt > 0
    iota_new = lax.broadcasted_iota(jnp.int32, (S, S), 0)
    p = jnp.where((ranks[None, :] == iota_new) & present[None, :], 1.0, 0.0)
    mean_new = lax.dot_general(p, mean, (((1,), (1,)), ((), ())),
                               preferred_element_type=jnp.float32)
    out = lax.dot_general(mean_new, pw_ref[...], (((1,), (1,)), ((), ())),
                          preferred_element_type=jnp.float32)
    out_ref[0] = out + pb_ref[...]


def kernel(x, segment, conv_w, conv_b, pos_embed, proj_w, proj_b):
    seg32 = segment.astype(jnp.int32)

    new_seg, counts, ranks, _ = _sc_reindex(seg32.reshape(B * HW))
    counts = counts.reshape(B, S)
    ranks = ranks.reshape(B, S)

    wh = jax.image.resize(jnp.eye(SRC, dtype=jnp.float32), (H, SRC), 'bilinear')
    ww = jax.image.resize(jnp.eye(SRC, dtype=jnp.float32), (SRC, W), 'bilinear')

    pos_hw = pl.pallas_call(
        _pos_body,
        grid=(NB,),
        in_specs=[
            pl.BlockSpec((CMID, SRC, SRC), lambda nb: (0, 0, 0)),
            pl.BlockSpec((R, SRC), lambda nb: (nb, 0)),
            pl.BlockSpec((SRC, W), lambda nb: (0, 0)),
        ],
        out_specs=pl.BlockSpec((CMID, R, W), lambda nb: (0, nb, 0)),
        out_shape=jax.ShapeDtypeStruct((CMID, H, W), jnp.float32),
        scratch_shapes=[pltpu.VMEM((CMID, SRC, W), jnp.float32)],
    )(pos_embed.reshape(CMID, SRC, SRC), wh, ww)

    acc = pl.pallas_call(
        _main_body,
        grid=(NB,),
        in_specs=[
            pl.BlockSpec((B, CIN, BLK), lambda nb: (0, 0, nb)),
            pl.BlockSpec((B, 1, BLK), lambda nb: (0, 0, nb)),
            pl.BlockSpec((CMID, BLK), lambda nb: (0, nb)),
            pl.BlockSpec((CIN, CMID), lambda nb: (0, 0)),
            pl.BlockSpec((CMID, 1), lambda nb: (0, 0)),
        ],
        out_specs=pl.BlockSpec((B, CMID, S), lambda nb: (0, 0, 0)),
        out_shape=jax.ShapeDtypeStruct((B, CMID, S), jnp.float32),
    )(x.reshape(B, CIN, HW), seg32.reshape(B, 1, HW),
      pos_hw.reshape(CMID, HW),
      conv_w.reshape(CMID, CIN).T, conv_b.reshape(CMID, 1))

    out = pl.pallas_call(
        _proj_body,
        grid=(B,),
        in_specs=[
            pl.BlockSpec((1, CMID, S), lambda b: (b, 0, 0)),
            pl.BlockSpec((1, 1, S), lambda b: (b, 0, 0)),
            pl.BlockSpec((1, 1, S), lambda b: (b, 0, 0)),
            pl.BlockSpec((COUT, CMID), lambda b: (0, 0)),
            pl.BlockSpec((1, COUT), lambda b: (0, 0)),
        ],
        out_specs=pl.BlockSpec((1, S, COUT), lambda b: (b, 0, 0)),
        out_shape=jax.ShapeDtypeStruct((B, S, COUT), jnp.float32),
    )(acc, counts.reshape(B, 1, S), ranks.reshape(B, 1, S),
      proj_w, proj_b.reshape(1, COUT))

    return (out, new_seg.reshape(B, H, W))


# speedup vs baseline: 1.4099x; 1.4099x over previous
import functools

import jax
import jax.numpy as jnp
from jax import lax
from jax.experimental import pallas as pl
from jax.experimental.pallas import tpu as pltpu
from jax.experimental.pallas import tpu_sc as plsc

B = 8
H = W = 224
HW = H * W
S = 1024
CIN = 3
CMID = 128
COUT = 768
SRC = 64
R = 8
BLK = R * W
NB = H // R
NTPB = 4
CH = HW // NTPB
NCH = CH // 16


def _sc_body(seg_hbm, newseg_hbm, counts_hbm, ranks_hbm, histall_hbm,
             ids_v, hist16_v, hist4_v, cnt_v, ranks_v, outbuf_v):
    c = lax.axis_index("c")
    s = lax.axis_index("s")
    b = c * 4 + s // 4
    q = s % 4
    wid = c * 16 + s
    base_row = wid - q

    pltpu.sync_copy(seg_hbm.at[pl.ds(b * HW + q * CH, CH)], ids_v)

    def _zero(i, _):
        hist16_v[pl.ds(i * 16, 16)] = jnp.zeros((16,), jnp.int32)
        return 0
    lax.fori_loop(0, 1024, _zero, 0)

    lane_off = lax.iota(jnp.int32, 16) * S
    ones16 = jnp.ones((16,), jnp.int32)

    def _hist(j, _):
        ids = ids_v[pl.ds(j * 16, 16)]
        plsc.addupdate_scatter(hist16_v, [ids + lane_off], ones16)
        return 0
    lax.fori_loop(0, NCH, _hist, 0)

    def _fold(i, _):
        acc = hist16_v[pl.ds(i * 16, 16)]
        for r in range(1, 16):
            acc = acc + hist16_v[pl.ds(r * S + i * 16, 16)]
        cnt_v[pl.ds(i * 16, 16)] = acc
        return 0
    lax.fori_loop(0, S // 16, _fold, 0)

    pltpu.sync_copy(cnt_v, histall_hbm.at[pl.ds(wid * S, S)])
    plsc.subcore_barrier()

    @pl.when(q == 0)
    def _phase2():
        pltpu.sync_copy(histall_hbm.at[pl.ds(base_row * S, 4 * S)], hist4_v)

        def _ranks(i, run):
            sl = pl.ds(i * 16, 16)
            h = (hist4_v[pl.ds(i * 16, 16)]
                 + hist4_v[pl.ds(S + i * 16, 16)]
                 + hist4_v[pl.ds(2 * S + i * 16, 16)]
                 + hist4_v[pl.ds(3 * S + i * 16, 16)])
            pres = jnp.where(h > 0, 1, 0).astype(jnp.int32)
            cs = plsc.cumsum(pres)
            ranks_v[sl] = cs - pres + run
            cnt_v[sl] = h
            return run + jnp.sum(pres)
        lax.fori_loop(0, S // 16, _ranks, jnp.int32(0))

        pltpu.sync_copy(ranks_v, ranks_hbm.at[pl.ds(b * S, S)])
        pltpu.sync_copy(cnt_v, counts_hbm.at[pl.ds(b * S, S)])

    plsc.subcore_barrier()

    pltpu.sync_copy(ranks_hbm.at[pl.ds(b * S, S)], ranks_v)

    def _gather(j, _):
        ids = ids_v[pl.ds(j * 16, 16)]
        outbuf_v[pl.ds(j * 16, 16)] = plsc.load_gather(ranks_v, [ids])
        return 0
    lax.fori_loop(0, NCH, _gather, 0)

    pltpu.sync_copy(outbuf_v, newseg_hbm.at[pl.ds(b * HW + q * CH, CH)])


def _sc_reindex(seg_flat):
    f = pl.kernel(
        _sc_body,
        mesh=plsc.VectorSubcoreMesh(core_axis_name="c", subcore_axis_name="s"),
        compiler_params=pltpu.CompilerParams(needs_layout_passes=False),
        out_type=[
            jax.ShapeDtypeStruct((B * HW,), jnp.int32),
            jax.ShapeDtypeStruct((B * S,), jnp.int32),
            jax.ShapeDtypeStruct((B * S,), jnp.int32),
            jax.ShapeDtypeStruct((32 * S,), jnp.int32),
        ],
        scratch_types=[
            pltpu.VMEM((CH,), jnp.int32),
            pltpu.VMEM((16 * S,), jnp.int32),
            pltpu.VMEM((4 * S,), jnp.int32),
            pltpu.VMEM((S,), jnp.int32),
            pltpu.VMEM((S,), jnp.int32),
            pltpu.VMEM((CH,), jnp.int32),
        ],
    )
    return f(seg_flat)


def _pos_body(pos_ref, wh_ref, ww_ref, out_ref, posw_scr):
    nb = pl.program_id(0)

    @pl.when(nb == 0)
    def _():
        posw_scr[...] = lax.dot_general(
            pos_ref[...], ww_ref[...], (((2,), (0,)), ((), ())),
            preferred_element_type=jnp.float32)

    whb = jnp.broadcast_to(wh_ref[...][None, :, :], (CMID, R, SRC))
    out_ref[...] = lax.dot_general(
        whb, posw_scr[...], (((2,), (1,)), ((0,), (0,))),
        preferred_element_type=jnp.float32)


def _main_body(x_ref, seg_ref, pos_ref, wc_ref, cb_ref, acc_ref):
    nb = pl.program_id(0)

    @pl.when(nb == 0)
    def _():
        acc_ref[...] = jnp.zeros_like(acc_ref)

    posr = pos_ref[...]
    iota_s = lax.broadcasted_iota(jnp.int16, (BLK, S), 1)
    one = jnp.bfloat16(1.0)
    zero = jnp.bfloat16(0.0)
    for b in range(B):
        xb = x_ref[b]
        feat = lax.dot_general(wc_ref[...], xb, (((0,), (0,)), ((), ())),
                               preferred_element_type=jnp.float32)
        v = jnp.maximum(feat + cb_ref[...], 0.0) + posr
        ids_t = lax.transpose(seg_ref[b], (1, 0)).astype(jnp.int16)
        ids_b = jnp.broadcast_to(ids_t, (BLK, S))
        oh = jnp.where(ids_b == iota_s, one, zero)
        acc_ref[b] = acc_ref[b] + lax.dot_general(
            v.astype(jnp.bfloat16), oh, (((1,), (0,)), ((), ())),
            preferred_element_type=jnp.float32)


def _proj_body(acc_ref, cnt_ref, rank_ref, pw_ref, pb_ref, out_ref):
    cnt = cnt_ref[0, 0, :]
    inv = 1.0 / jnp.maximum(cnt.astype(jnp.float32), 1.0)
    mean = acc_ref[0] * inv[None, :]

    ranks = rank_ref[0, 0, :]
    present = cnt > 0
    iota_new = lax.broadcasted_iota(jnp.int32, (S, S), 0)
    p = jnp.where((ranks[None, :] == iota_new) & present[None, :], 1.0, 0.0)
    mean_new = lax.dot_general(p, mean, (((1,), (1,)), ((), ())),
                               preferred_element_type=jnp.float32)
    out = lax.dot_general(mean_new, pw_ref[...], (((1,), (1,)), ((), ())),
                          preferred_element_type=jnp.float32)
    out_ref[0] = out + pb_ref[...]


def kernel(x, segment, conv_w, conv_b, pos_embed, proj_w, proj_b):
    seg32 = segment.astype(jnp.int32)

    new_seg, counts, ranks, _ = _sc_reindex(seg32.reshape(B * HW))
    counts = counts.reshape(B, S)
    ranks = ranks.reshape(B, S)

    wh = jax.image.resize(jnp.eye(SRC, dtype=jnp.float32), (H, SRC), 'bilinear')
    ww = jax.image.resize(jnp.eye(SRC, dtype=jnp.float32), (SRC, W), 'bilinear')

    pos_hw = pl.pallas_call(
        _pos_body,
        grid=(NB,),
        in_specs=[
            pl.BlockSpec((CMID, SRC, SRC), lambda nb: (0, 0, 0)),
            pl.BlockSpec((R, SRC), lambda nb: (nb, 0)),
            pl.BlockSpec((SRC, W), lambda nb: (0, 0)),
        ],
        out_specs=pl.BlockSpec((CMID, R, W), lambda nb: (0, nb, 0)),
        out_shape=jax.ShapeDtypeStruct((CMID, H, W), jnp.float32),
        scratch_shapes=[pltpu.VMEM((CMID, SRC, W), jnp.float32)],
    )(pos_embed.reshape(CMID, SRC, SRC), wh, ww)

    acc = pl.pallas_call(
        _main_body,
        grid=(NB,),
        in_specs=[
            pl.BlockSpec((B, CIN, BLK), lambda nb: (0, 0, nb)),
            pl.BlockSpec((B, 1, BLK), lambda nb: (0, 0, nb)),
            pl.BlockSpec((CMID, BLK), lambda nb: (0, nb)),
            pl.BlockSpec((CIN, CMID), lambda nb: (0, 0)),
            pl.BlockSpec((CMID, 1), lambda nb: (0, 0)),
        ],
        out_specs=pl.BlockSpec((B, CMID, S), lambda nb: (0, 0, 0)),
        out_shape=jax.ShapeDtypeStruct((B, CMID, S), jnp.float32),
    )(x.reshape(B, CIN, HW), seg32.reshape(B, 1, HW),
      pos_hw.reshape(CMID, HW),
      conv_w.reshape(CMID, CIN).T, conv_b.reshape(CMID, 1))

    out = pl.pallas_call(
        _proj_body,
        grid=(B,),
        in_specs=[
            pl.BlockSpec((1, CMID, S), lambda b: (b, 0, 0)),
            pl.BlockSpec((1, 1, S), lambda b: (b, 0, 0)),
            pl.BlockSpec((1, 1, S), lambda b: (b, 0, 0)),
            pl.BlockSpec((COUT, CMID), lambda b: (0, 0)),
            pl.BlockSpec((1, COUT), lambda b: (0, 0)),
        ],
        out_specs=pl.BlockSpec((1, S, COUT), lambda b: (b, 0, 0)),
        out_shape=jax.ShapeDtypeStruct((B, S, COUT), jnp.float32),
    )(acc, counts.reshape(B, 1, S), ranks.reshape(B, 1, S),
      proj_w, proj_b.reshape(1, COUT))

    return (out, new_seg.reshape(B, H, W))


# speedup vs baseline: 1.4953x; 1.0606x over previous
import functools

import jax
import jax.numpy as jnp
from jax import lax
from jax.experimental import pallas as pl
from jax.experimental.pallas import tpu as pltpu
from jax.experimental.pallas import tpu_sc as plsc

B = 8
H = W = 224
HW = H * W
S = 1024
CIN = 3
CMID = 128
COUT = 768
SRC = 64
R = 8
BLK = R * W
NB = H // R
NTPB = 4
CH = HW // NTPB
NCH = CH // 16


def _sc_body(seg_hbm, newseg_hbm, counts_hbm, ranks_hbm, histall_hbm,
             ids_v, hist16_v, hist4_v, cnt_v, ranks_v, outbuf_v):
    c = lax.axis_index("c")
    s = lax.axis_index("s")
    b = c * 4 + s // 4
    q = s % 4
    wid = c * 16 + s
    base_row = wid - q

    pltpu.sync_copy(seg_hbm.at[pl.ds(b * HW + q * CH, CH)], ids_v)

    def _zero(i, _):
        for k in range(8):
            hist16_v[pl.ds((i * 8 + k) * 16, 16)] = jnp.zeros((16,), jnp.int32)
        return 0
    lax.fori_loop(0, 128, _zero, 0)

    lane_off = lax.iota(jnp.int32, 16) * S
    ones16 = jnp.ones((16,), jnp.int32)

    def _hist(j, _):
        for k in range(8):
            ids = ids_v[pl.ds((j * 8 + k) * 16, 16)]
            plsc.addupdate_scatter(hist16_v, [ids + lane_off], ones16)
        return 0
    lax.fori_loop(0, NCH // 8, _hist, 0)

    def _fold(i, _):
        acc = hist16_v[pl.ds(i * 16, 16)]
        for r in range(1, 16):
            acc = acc + hist16_v[pl.ds(r * S + i * 16, 16)]
        cnt_v[pl.ds(i * 16, 16)] = acc
        return 0
    lax.fori_loop(0, S // 16, _fold, 0)

    pltpu.sync_copy(cnt_v, histall_hbm.at[pl.ds(wid * S, S)])
    plsc.subcore_barrier()

    @pl.when(q == 0)
    def _phase2():
        pltpu.sync_copy(histall_hbm.at[pl.ds(base_row * S, 4 * S)], hist4_v)

        def _ranks(i, run):
            sl = pl.ds(i * 16, 16)
            h = (hist4_v[pl.ds(i * 16, 16)]
                 + hist4_v[pl.ds(S + i * 16, 16)]
                 + hist4_v[pl.ds(2 * S + i * 16, 16)]
                 + hist4_v[pl.ds(3 * S + i * 16, 16)])
            pres = jnp.where(h > 0, 1, 0).astype(jnp.int32)
            cs = plsc.cumsum(pres)
            ranks_v[sl] = cs - pres + run
            cnt_v[sl] = h
            return run + jnp.sum(pres)
        lax.fori_loop(0, S // 16, _ranks, jnp.int32(0))

        pltpu.sync_copy(ranks_v, ranks_hbm.at[pl.ds(b * S, S)])
        pltpu.sync_copy(cnt_v, counts_hbm.at[pl.ds(b * S, S)])

    plsc.subcore_barrier()

    pltpu.sync_copy(ranks_hbm.at[pl.ds(b * S, S)], ranks_v)

    def _gather(j, _):
        for k in range(8):
            sl = pl.ds((j * 8 + k) * 16, 16)
            outbuf_v[sl] = plsc.load_gather(ranks_v, [ids_v[sl]])
        return 0
    lax.fori_loop(0, NCH // 8, _gather, 0)

    pltpu.sync_copy(outbuf_v, newseg_hbm.at[pl.ds(b * HW + q * CH, CH)])


def _sc_reindex(seg_flat):
    f = pl.kernel(
        _sc_body,
        mesh=plsc.VectorSubcoreMesh(core_axis_name="c", subcore_axis_name="s"),
        compiler_params=pltpu.CompilerParams(needs_layout_passes=False),
        out_type=[
            jax.ShapeDtypeStruct((B * HW,), jnp.int32),
            jax.ShapeDtypeStruct((B * S,), jnp.int32),
            jax.ShapeDtypeStruct((B * S,), jnp.int32),
            jax.ShapeDtypeStruct((32 * S,), jnp.int32),
        ],
        scratch_types=[
            pltpu.VMEM((CH,), jnp.int32),
            pltpu.VMEM((16 * S,), jnp.int32),
            pltpu.VMEM((4 * S,), jnp.int32),
            pltpu.VMEM((S,), jnp.int32),
            pltpu.VMEM((S,), jnp.int32),
            pltpu.VMEM((CH,), jnp.int32),
        ],
    )
    return f(seg_flat)


def _pos_body(pos_ref, wh_ref, ww_ref, out_ref, posw_scr):
    nb = pl.program_id(0)

    @pl.when(nb == 0)
    def _():
        posw_scr[...] = lax.dot_general(
            pos_ref[...], ww_ref[...], (((2,), (0,)), ((), ())),
            preferred_element_type=jnp.float32)

    whb = jnp.broadcast_to(wh_ref[...][None, :, :], (CMID, R, SRC))
    out_ref[...] = lax.dot_general(
        whb, posw_scr[...], (((2,), (1,)), ((0,), (0,))),
        preferred_element_type=jnp.float32).astype(jnp.bfloat16)


def _main_body(x_ref, seg_ref, pos_ref, wc_ref, cb_ref, acc_ref):
    nb = pl.program_id(0)

    @pl.when(nb == 0)
    def _():
        acc_ref[...] = jnp.zeros_like(acc_ref)

    posr = pos_ref[...]
    iota_s = lax.broadcasted_iota(jnp.int16, (BLK, S), 1)
    one = jnp.bfloat16(1.0)
    zero = jnp.bfloat16(0.0)
    for b in range(B):
        xb = x_ref[b]
        feat = lax.dot_general(wc_ref[...], xb, (((0,), (0,)), ((), ())),
                               preferred_element_type=jnp.float32)
        v = jnp.maximum(feat + cb_ref[...], 0.0).astype(jnp.bfloat16) + posr
        ids_t = lax.transpose(seg_ref[b], (1, 0)).astype(jnp.int16)
        ids_b = jnp.broadcast_to(ids_t, (BLK, S))
        oh = jnp.where(ids_b == iota_s, one, zero)
        acc_ref[b] = acc_ref[b] + lax.dot_general(
            v, oh, (((1,), (0,)), ((), ())),
            preferred_element_type=jnp.float32)


def _proj_body(acc_ref, cnt_ref, rank_ref, pw_ref, pb_ref, out_ref):
    cnt = cnt_ref[0, 0, :]
    inv = 1.0 / jnp.maximum(cnt.astype(jnp.float32), 1.0)
    mean = acc_ref[0] * inv[None, :]

    ranks = rank_ref[0, 0, :]
    present = cnt > 0
    iota_new = lax.broadcasted_iota(jnp.int32, (S, S), 0)
    p = jnp.where((ranks[None, :] == iota_new) & present[None, :], 1.0, 0.0)
    mean_new = lax.dot_general(p, mean, (((1,), (1,)), ((), ())),
                               preferred_element_type=jnp.float32)
    out = lax.dot_general(mean_new, pw_ref[...], (((1,), (1,)), ((), ())),
                          preferred_element_type=jnp.float32)
    out_ref[0] = out + pb_ref[...]


def kernel(x, segment, conv_w, conv_b, pos_embed, proj_w, proj_b):
    seg32 = segment.astype(jnp.int32)

    new_seg, counts, ranks, _ = _sc_reindex(seg32.reshape(B * HW))
    counts = counts.reshape(B, S)
    ranks = ranks.reshape(B, S)

    wh = jax.image.resize(jnp.eye(SRC, dtype=jnp.float32), (H, SRC), 'bilinear')
    ww = jax.image.resize(jnp.eye(SRC, dtype=jnp.float32), (SRC, W), 'bilinear')

    pos_hw = pl.pallas_call(
        _pos_body,
        grid=(NB,),
        in_specs=[
            pl.BlockSpec((CMID, SRC, SRC), lambda nb: (0, 0, 0)),
            pl.BlockSpec((R, SRC), lambda nb: (nb, 0)),
            pl.BlockSpec((SRC, W), lambda nb: (0, 0)),
        ],
        out_specs=pl.BlockSpec((CMID, R, W), lambda nb: (0, nb, 0)),
        out_shape=jax.ShapeDtypeStruct((CMID, H, W), jnp.bfloat16),
        scratch_shapes=[pltpu.VMEM((CMID, SRC, W), jnp.float32)],
    )(pos_embed.reshape(CMID, SRC, SRC), wh, ww)

    acc = pl.pallas_call(
        _main_body,
        grid=(NB,),
        in_specs=[
            pl.BlockSpec((B, CIN, BLK), lambda nb: (0, 0, nb)),
            pl.BlockSpec((B, 1, BLK), lambda nb: (0, 0, nb)),
            pl.BlockSpec((CMID, BLK), lambda nb: (0, nb)),
            pl.BlockSpec((CIN, CMID), lambda nb: (0, 0)),
            pl.BlockSpec((CMID, 1), lambda nb: (0, 0)),
        ],
        out_specs=pl.BlockSpec((B, CMID, S), lambda nb: (0, 0, 0)),
        out_shape=jax.ShapeDtypeStruct((B, CMID, S), jnp.float32),
    )(x.reshape(B, CIN, HW), seg32.reshape(B, 1, HW),
      pos_hw.reshape(CMID, HW),
      conv_w.reshape(CMID, CIN).T, conv_b.reshape(CMID, 1))

    out = pl.pallas_call(
        _proj_body,
        grid=(B,),
        in_specs=[
            pl.BlockSpec((1, CMID, S), lambda b: (b, 0, 0)),
            pl.BlockSpec((1, 1, S), lambda b: (b, 0, 0)),
            pl.BlockSpec((1, 1, S), lambda b: (b, 0, 0)),
            pl.BlockSpec((COUT, CMID), lambda b: (0, 0)),
            pl.BlockSpec((1, COUT), lambda b: (0, 0)),
        ],
        out_specs=pl.BlockSpec((1, S, COUT), lambda b: (b, 0, 0)),
        out_shape=jax.ShapeDtypeStruct((B, S, COUT), jnp.float32),
    )(acc, counts.reshape(B, 1, S), ranks.reshape(B, 1, S),
      proj_w, proj_b.reshape(1, COUT))

    return (out, new_seg.reshape(B, H, W))


# speedup vs baseline: 1.5715x; 1.0510x over previous
import functools

import jax
import jax.numpy as jnp
from jax import lax
from jax.experimental import pallas as pl
from jax.experimental.pallas import tpu as pltpu
from jax.experimental.pallas import tpu_sc as plsc

B = 8
H = W = 224
HW = H * W
S = 1024
CIN = 3
CMID = 128
COUT = 768
SRC = 64
R = 8
BLK = R * W
NB = H // R
NTPB = 4
CH = HW // NTPB
NCH = CH // 16
RP = 32
NBP = H // RP


def _sc_body(seg_hbm, newseg_hbm, counts_hbm, ranks_hbm, histall_hbm,
             ids_v, hist16_v, hist4_v, cnt_v, ranks_v, outbuf_v):
    c = lax.axis_index("c")
    s = lax.axis_index("s")
    b = c * 4 + s // 4
    q = s % 4
    wid = c * 16 + s
    base_row = wid - q

    pltpu.sync_copy(seg_hbm.at[pl.ds(b * HW + q * CH, CH)], ids_v)

    def _zero(i, _):
        for k in range(8):
            hist16_v[pl.ds((i * 8 + k) * 16, 16)] = jnp.zeros((16,), jnp.int32)
        return 0
    lax.fori_loop(0, 128, _zero, 0)

    lane_off = lax.iota(jnp.int32, 16) * S
    ones16 = jnp.ones((16,), jnp.int32)

    def _hist(j, _):
        for k in range(8):
            ids = ids_v[pl.ds((j * 8 + k) * 16, 16)]
            plsc.addupdate_scatter(hist16_v, [ids + lane_off], ones16)
        return 0
    lax.fori_loop(0, NCH // 8, _hist, 0)

    def _fold(i, _):
        acc = hist16_v[pl.ds(i * 16, 16)]
        for r in range(1, 16):
            acc = acc + hist16_v[pl.ds(r * S + i * 16, 16)]
        cnt_v[pl.ds(i * 16, 16)] = acc
        return 0
    lax.fori_loop(0, S // 16, _fold, 0)

    pltpu.sync_copy(cnt_v, histall_hbm.at[pl.ds(wid * S, S)])
    plsc.subcore_barrier()

    @pl.when(q == 0)
    def _phase2():
        pltpu.sync_copy(histall_hbm.at[pl.ds(base_row * S, 4 * S)], hist4_v)

        def _ranks(i, run):
            sl = pl.ds(i * 16, 16)
            h = (hist4_v[pl.ds(i * 16, 16)]
                 + hist4_v[pl.ds(S + i * 16, 16)]
                 + hist4_v[pl.ds(2 * S + i * 16, 16)]
                 + hist4_v[pl.ds(3 * S + i * 16, 16)])
            pres = jnp.where(h > 0, 1, 0).astype(jnp.int32)
            cs = plsc.cumsum(pres)
            ranks_v[sl] = cs - pres + run
            cnt_v[sl] = h
            return run + jnp.sum(pres)
        lax.fori_loop(0, S // 16, _ranks, jnp.int32(0))

        pltpu.sync_copy(ranks_v, ranks_hbm.at[pl.ds(b * S, S)])
        pltpu.sync_copy(cnt_v, counts_hbm.at[pl.ds(b * S, S)])

    plsc.subcore_barrier()

    pltpu.sync_copy(ranks_hbm.at[pl.ds(b * S, S)], ranks_v)

    def _gather(j, _):
        for k in range(8):
            sl = pl.ds((j * 8 + k) * 16, 16)
            outbuf_v[sl] = plsc.load_gather(ranks_v, [ids_v[sl]])
        return 0
    lax.fori_loop(0, NCH // 8, _gather, 0)

    pltpu.sync_copy(outbuf_v, newseg_hbm.at[pl.ds(b * HW + q * CH, CH)])


def _sc_reindex(seg_flat):
    f = pl.kernel(
        _sc_body,
        mesh=plsc.VectorSubcoreMesh(core_axis_name="c", subcore_axis_name="s"),
        compiler_params=pltpu.CompilerParams(needs_layout_passes=False),
        out_type=[
            jax.ShapeDtypeStruct((B * HW,), jnp.int32),
            jax.ShapeDtypeStruct((B * S,), jnp.int32),
            jax.ShapeDtypeStruct((B * S,), jnp.int32),
            jax.ShapeDtypeStruct((32 * S,), jnp.int32),
        ],
        scratch_types=[
            pltpu.VMEM((CH,), jnp.int32),
            pltpu.VMEM((16 * S,), jnp.int32),
            pltpu.VMEM((4 * S,), jnp.int32),
            pltpu.VMEM((S,), jnp.int32),
            pltpu.VMEM((S,), jnp.int32),
            pltpu.VMEM((CH,), jnp.int32),
        ],
    )
    return f(seg_flat)


def _pos_body(pos_ref, wh_ref, ww_ref, out_ref, posw_scr):
    nb = pl.program_id(0)

    @pl.when(nb == 0)
    def _():
        posw_scr[...] = lax.dot_general(
            pos_ref[...], ww_ref[...], (((2,), (0,)), ((), ())),
            preferred_element_type=jnp.float32)

    whb = jnp.broadcast_to(wh_ref[...][None, :, :], (CMID, RP, SRC))
    out_ref[...] = lax.dot_general(
        whb, posw_scr[...], (((2,), (1,)), ((0,), (0,))),
        preferred_element_type=jnp.float32).astype(jnp.bfloat16)


def _main_body(x_ref, seg_ref, pos_ref, wc_ref, cb_ref, acc_ref):
    nb = pl.program_id(0)

    @pl.when(nb == 0)
    def _():
        acc_ref[...] = jnp.zeros_like(acc_ref)

    posr = pos_ref[...]
    iota_s = lax.broadcasted_iota(jnp.int16, (BLK, S), 1)
    one = jnp.bfloat16(1.0)
    zero = jnp.bfloat16(0.0)
    for b in range(B):
        xb = x_ref[b]
        feat = lax.dot_general(wc_ref[...], xb, (((0,), (0,)), ((), ())),
                               preferred_element_type=jnp.float32)
        v = jnp.maximum(feat + cb_ref[...], 0.0).astype(jnp.bfloat16) + posr
        ids_t = lax.transpose(seg_ref[b], (1, 0)).astype(jnp.int16)
        ids_b = jnp.broadcast_to(ids_t, (BLK, S))
        oh = jnp.where(ids_b == iota_s, one, zero)
        acc_ref[b] = acc_ref[b] + lax.dot_general(
            v, oh, (((1,), (0,)), ((), ())),
            preferred_element_type=jnp.float32)


def _proj_body(acc_ref, cnt_ref, rank_ref, pw_ref, pb_ref, out_ref):
    cnt = cnt_ref[0, 0, :]
    inv = 1.0 / jnp.maximum(cnt.astype(jnp.float32), 1.0)
    mean = acc_ref[0] * inv[None, :]

    ranks = rank_ref[0, 0, :]
    present = cnt > 0
    iota_new = lax.broadcasted_iota(jnp.int32, (S, S), 0)
    p = jnp.where((ranks[None, :] == iota_new) & present[None, :], 1.0, 0.0)
    mean_new = lax.dot_general(p, mean, (((1,), (1,)), ((), ())),
                               preferred_element_type=jnp.float32)
    out = lax.dot_general(mean_new, pw_ref[...], (((1,), (1,)), ((), ())),
                          preferred_element_type=jnp.float32)
    out_ref[0] = out + pb_ref[...]


def kernel(x, segment, conv_w, conv_b, pos_embed, proj_w, proj_b):
    seg32 = segment.astype(jnp.int32)

    new_seg, counts, ranks, _ = _sc_reindex(seg32.reshape(B * HW))
    counts = counts.reshape(B, S)
    ranks = ranks.reshape(B, S)

    wh = jax.image.resize(jnp.eye(SRC, dtype=jnp.float32), (H, SRC), 'bilinear')
    ww = jax.image.resize(jnp.eye(SRC, dtype=jnp.float32), (SRC, W), 'bilinear')

    pos_hw = pl.pallas_call(
        _pos_body,
        grid=(NBP,),
        in_specs=[
            pl.BlockSpec((CMID, SRC, SRC), lambda nb: (0, 0, 0)),
            pl.BlockSpec((RP, SRC), lambda nb: (nb, 0)),
            pl.BlockSpec((SRC, W), lambda nb: (0, 0)),
        ],
        out_specs=pl.BlockSpec((CMID, RP, W), lambda nb: (0, nb, 0)),
        out_shape=jax.ShapeDtypeStruct((CMID, H, W), jnp.bfloat16),
        scratch_shapes=[pltpu.VMEM((CMID, SRC, W), jnp.float32)],
    )(pos_embed.reshape(CMID, SRC, SRC), wh, ww)

    acc = pl.pallas_call(
        _main_body,
        grid=(NB,),
        in_specs=[
            pl.BlockSpec((B, CIN, BLK), lambda nb: (0, 0, nb)),
            pl.BlockSpec((B, 1, BLK), lambda nb: (0, 0, nb)),
            pl.BlockSpec((CMID, BLK), lambda nb: (0, nb)),
            pl.BlockSpec((CIN, CMID), lambda nb: (0, 0)),
            pl.BlockSpec((CMID, 1), lambda nb: (0, 0)),
        ],
        out_specs=pl.BlockSpec((B, CMID, S), lambda nb: (0, 0, 0)),
        out_shape=jax.ShapeDtypeStruct((B, CMID, S), jnp.float32),
    )(x.reshape(B, CIN, HW), seg32.reshape(B, 1, HW),
      pos_hw.reshape(CMID, HW),
      conv_w.reshape(CMID, CIN).T, conv_b.reshape(CMID, 1))

    out = pl.pallas_call(
        _proj_body,
        grid=(B,),
        in_specs=[
            pl.BlockSpec((1, CMID, S), lambda b: (b, 0, 0)),
            pl.BlockSpec((1, 1, S), lambda b: (b, 0, 0)),
            pl.BlockSpec((1, 1, S), lambda b: (b, 0, 0)),
            pl.BlockSpec((COUT, CMID), lambda b: (0, 0)),
            pl.BlockSpec((1, COUT), lambda b: (0, 0)),
        ],
        out_specs=pl.BlockSpec((1, S, COUT), lambda b: (b, 0, 0)),
        out_shape=jax.ShapeDtypeStruct((B, S, COUT), jnp.float32),
    )(acc, counts.reshape(B, 1, S), ranks.reshape(B, 1, S),
      proj_w, proj_b.reshape(1, COUT))

    return (out, new_seg.reshape(B, H, W))
